# baseline (speedup 1.0000x reference)
import functools

import jax
import jax.numpy as jnp
from jax import lax
from jax.experimental import pallas as pl
from jax.experimental.pallas import tpu as pltpu
from jax.experimental.pallas import tpu_sc as plsc

G = 2
V = 320
D = 512
K = 128
TILE = 1024

NC = 2
NS = 16
NW = NC * NS
CH = 512


def _tc_body(x_ref, mrow_ref, w_ref, b_ref, fidx_ref, ppl_ref,
             counts_ref, idx_ref, *, n_steps):
    i = pl.program_id(0)

    @pl.when(i == 0)
    def _init():
        counts_ref[...] = jnp.zeros_like(counts_ref)

    idx_prev = [idx_ref[g][...] for g in range(G)]

    x = x_ref[...]
    iota = lax.broadcasted_iota(jnp.int32, (TILE, V), 1).astype(jnp.float32)
    cols = []
    for g in range(G):
        logits = lax.dot_general(
            x, w_ref[g], (((1,), (1,)), ((), ())),
            preferred_element_type=jnp.float32)
        logits = logits + b_ref[g][None, :]
        mx = jnp.max(logits, axis=-1, keepdims=True)
        cand = jnp.where(logits == mx, iota, float(V))
        idxf = jnp.min(cand, axis=-1, keepdims=True)
        idx_ref[g] = idxf
        cols.append(idxf.astype(jnp.int32) + g * V)
    fidx_ref[...] = jnp.concatenate(cols, axis=1)

    mrow = mrow_ref[0]
    live = jnp.where(i > 0, 1.0, 0.0)
    for g in range(G):
        oh = (iota == idx_prev[g]).astype(jnp.float32)
        counts_ref[g:g + 1, :] += live * lax.dot_general(
            mrow, oh, (((1,), (0,)), ((), ())),
            preferred_element_type=jnp.float32)

    @pl.when(i == n_steps)
    def _finalize():
        counts = counts_ref[...]
        denom = jnp.maximum(jnp.sum(counts) * (1.0 / G), 1.0)
        avg = counts / denom
        plogp = avg * jnp.log(avg + 1e-07)
        neg = -jnp.sum(plogp, axis=1, keepdims=True)
        ppl_ref[...] = jnp.sum(jnp.exp(neg), axis=0, keepdims=True)


def _sc_gather_body(table_ref, fidx_ref, out_ref, idx_v, rows_v, sem,
                    *, n_ch):
    wid = lax.axis_index("s") * NC + lax.axis_index("c")
    base = wid * (n_ch * CH)
    for c in range(n_ch):
        off = base + c * CH
        pltpu.sync_copy(fidx_ref.at[pl.ds(off, CH)], idx_v)
        pltpu.async_copy(table_ref.at[idx_v], rows_v, sem).wait()
        pltpu.sync_copy(rows_v, out_ref.at[pl.ds(off, CH)])


def kernel(hidden_states, mask_time_indices, codevectors, W, b):
    B, L, Dd = hidden_states.shape
    N = B * L
    n_steps = N // TILE
    x = hidden_states.reshape(N, Dd)
    mrow = mask_time_indices.reshape(n_steps, 1, TILE).astype(jnp.float32)
    w3 = W.reshape(G, V, Dd)
    b2 = b.reshape(G, V)
    table = codevectors.reshape(G * V, K)

    fidx, ppl = pl.pallas_call(
        functools.partial(_tc_body, n_steps=n_steps),
        grid=(n_steps + 1,),
        in_specs=[
            pl.BlockSpec((TILE, Dd), lambda i: (jnp.minimum(i, n_steps - 1), 0)),
            pl.BlockSpec((1, 1, TILE), lambda i: (jnp.maximum(i - 1, 0), 0, 0)),
            pl.BlockSpec((G, V, Dd), lambda i: (0, 0, 0)),
            pl.BlockSpec((G, V), lambda i: (0, 0)),
        ],
        out_specs=[
            pl.BlockSpec((TILE, G), lambda i: (jnp.minimum(i, n_steps - 1), 0)),
            pl.BlockSpec((1, 1), lambda i: (0, 0)),
        ],
        out_shape=[
            jax.ShapeDtypeStruct((N, G), jnp.int32),
            jax.ShapeDtypeStruct((1, 1), jnp.float32),
        ],
        scratch_shapes=[
            pltpu.VMEM((G, V), jnp.float32),
            pltpu.VMEM((G, TILE, 1), jnp.float32),
        ],
    )(x, mrow, w3, b2)

    n_rows = N * G
    n_ch = n_rows // (NW * CH)
    sc_gather = functools.partial(
        pl.kernel,
        out_type=jax.ShapeDtypeStruct((n_rows, K), jnp.float32),
        mesh=plsc.VectorSubcoreMesh(core_axis_name="c", subcore_axis_name="s"),
        scratch_types=[
            pltpu.VMEM((CH,), jnp.int32),
            pltpu.VMEM((CH, K), jnp.float32),
            pltpu.SemaphoreType.DMA,
        ],
    )(functools.partial(_sc_gather_body, n_ch=n_ch))
    out = sc_gather(table, fidx.reshape(n_rows))

    return out.reshape(B, L, G * K), ppl.reshape(())


# speedup vs baseline: 2.3950x; 2.3950x over previous
import functools

import jax
import jax.numpy as jnp
from jax import lax
from jax.experimental import pallas as pl
from jax.experimental.pallas import tpu as pltpu

G = 2
V = 320
D = 512
K = 128
TILE = 2048


def _body(x_ref, mrow_ref, w_ref, b_ref, cb_ref, out_ref, ppl_ref,
          counts_ref, idx_ref, *, n_steps):
    i = pl.program_id(0)

    @pl.when(i == 0)
    def _init():
        counts_ref[...] = jnp.zeros_like(counts_ref)

    idx_prev = [idx_ref[g][...] for g in range(G)]

    x = x_ref[...]
    iota = lax.broadcasted_iota(jnp.int32, (TILE, V), 1).astype(jnp.float32)
    for g in range(G):
        logits = lax.dot_general(
            x, w_ref[g], (((1,), (1,)), ((), ())),
            preferred_element_type=jnp.float32)
        logits = logits + b_ref[g][None, :]
        mx = jnp.max(logits, axis=-1, keepdims=True)
        cand = jnp.where(logits == mx, iota, float(V))
        idx_ref[g] = jnp.min(cand, axis=-1, keepdims=True)

    mrow = mrow_ref[0]
    live = jnp.where(i > 0, 1.0, 0.0)
    for g in range(G):
        oh = (iota == idx_prev[g]).astype(jnp.float32)
        counts_ref[g:g + 1, :] += live * lax.dot_general(
            mrow, oh, (((1,), (0,)), ((), ())),
            preferred_element_type=jnp.float32)
        out_ref[:, g * K:(g + 1) * K] = lax.dot_general(
            oh, cb_ref[g], (((1,), (0,)), ((), ())),
            preferred_element_type=jnp.float32)

    @pl.when(i == n_steps)
    def _finalize():
        counts = counts_ref[...]
        denom = jnp.maximum(jnp.sum(counts) * (1.0 / G), 1.0)
        avg = counts / denom
        plogp = avg * jnp.log(avg + 1e-07)
        neg = -jnp.sum(plogp, axis=1, keepdims=True)
        ppl_ref[...] = jnp.sum(jnp.exp(neg), axis=0, keepdims=True)


def kernel(hidden_states, mask_time_indices, codevectors, W, b):
    B, L, Dd = hidden_states.shape
    N = B * L
    n_steps = N // TILE
    x = hidden_states.reshape(N, Dd)
    mrow = mask_time_indices.reshape(n_steps, 1, TILE).astype(jnp.float32)
    w3 = W.reshape(G, V, Dd)
    b2 = b.reshape(G, V)
    cb = codevectors.reshape(G, V, K)

    out, ppl = pl.pallas_call(
        functools.partial(_body, n_steps=n_steps),
        grid=(n_steps + 1,),
        in_specs=[
            pl.BlockSpec((TILE, Dd), lambda i: (jnp.minimum(i, n_steps - 1), 0)),
            pl.BlockSpec((1, 1, TILE), lambda i: (jnp.maximum(i - 1, 0), 0, 0)),
            pl.BlockSpec((G, V, Dd), lambda i: (0, 0, 0)),
            pl.BlockSpec((G, V), lambda i: (0, 0)),
            pl.BlockSpec((G, V, K), lambda i: (0, 0, 0)),
        ],
        out_specs=[
            pl.BlockSpec((TILE, G * K), lambda i: (jnp.maximum(i - 1, 0), 0)),
            pl.BlockSpec((1, 1), lambda i: (0, 0)),
        ],
        out_shape=[
            jax.ShapeDtypeStruct((N, G * K), jnp.float32),
            jax.ShapeDtypeStruct((1, 1), jnp.float32),
        ],
        scratch_shapes=[
            pltpu.VMEM((G, V), jnp.float32),
            pltpu.VMEM((G, TILE, 1), jnp.float32),
        ],
    )(x, mrow, w3, b2, cb)

    return out.reshape(B, L, G * K), ppl.reshape(())
